# Initial kernel scaffold; baseline (speedup 1.0000x reference)
import jax
import jax.numpy as jnp
from jax.experimental import pallas as pl


def kernel(x, single_w, pair_w):
    raise NotImplementedError("write your pallas kernel here")


# speedup vs baseline: 6.6911x; 6.6911x over previous
import functools

import jax
import jax.numpy as jnp
import numpy as np
from jax.experimental import pallas as pl
from jax.experimental.pallas import tpu as pltpu

L = 128
V = 20
B = 64
D = L * V
BLK = 512
NBLK = D // BLK
BP = 72


def _mrf_kernel(a_ref, xf_ref, xb_ref, swb_ref, swf_ref, out_ref,
                se_acc, en_acc, r2_acc, r1_acc):
    ib = pl.program_id(0)
    a = a_ref[...]
    xf = xf_ref[...]
    xb = xb_ref[...]
    swb = swb_ref[...]

    nt = (((1,), (1,)), ((), ()))
    s_tile = jax.lax.dot_general(xf, a, nt, preferred_element_type=jnp.float32)

    ri = jax.lax.broadcasted_iota(jnp.int32, (BLK, D), 0) + ib * BLK
    ci = jax.lax.broadcasted_iota(jnp.int32, (BLK, D), 1)
    i_idx = ri // V
    l_idx = ci // V
    a_masked = jnp.where(l_idx <= i_idx, a, 0.0)
    r_tile = jax.lax.dot_general(xf, a_masked, nt,
                                 preferred_element_type=jnp.float32)

    coef = (ci == ri).astype(jnp.float32) - (ci == i_idx * V + 19).astype(
        jnp.float32)
    dcorr = jnp.sum(a * coef, axis=1)[None, :]

    c_row = r_tile[B:B + 1, :] + dcorr
    sap = c_row + s_tile - r_tile

    gi = jax.lax.broadcasted_iota(jnp.int32, (BLK, L), 0) + ib * BLK
    gj = jax.lax.broadcasted_iota(jnp.int32, (BLK, L), 1)
    g = (gi // V == gj).astype(jnp.float32)

    nn = (((1,), (0,)), ((), ()))
    e_blk = jnp.exp(swb + sap)
    se_part = jax.lax.dot_general(e_blk, g, nn,
                                  preferred_element_type=jnp.float32)
    en_part = jax.lax.dot_general((swb + s_tile) * xb, g, nn,
                                  preferred_element_type=jnp.float32)

    r2_part = jnp.sum(a * a)
    r1_part = jnp.sum(jnp.abs(a))

    @pl.when(ib == 0)
    def _init():
        se_acc[...] = se_part
        en_acc[...] = en_part
        r2_acc[0, 0] = r2_part
        r1_acc[0, 0] = r1_part

    @pl.when(ib > 0)
    def _accum():
        se_acc[...] += se_part
        en_acc[...] += en_part
        r2_acc[0, 0] += r2_part
        r1_acc[0, 0] += r1_part

    @pl.when(ib == NBLK - 1)
    def _finalize():
        te = jnp.log(se_acc[...])
        diff = te - en_acc[...]
        rowmask = jax.lax.broadcasted_iota(jnp.int32, (BP, L), 0) < B
        final_energy = jnp.sum(jnp.where(rowmask, diff, 0.0)) / B
        swf = swf_ref[...]
        lam_p = np.float32(0.2 * (L - 1))
        loss = (final_energy
                + jnp.sum(swf * swf) + jnp.sum(jnp.abs(swf))
                + lam_p * (r2_acc[0, 0] + r1_acc[0, 0]))
        out_ref[0, 0] = loss


@functools.partial(jax.jit, static_argnames=())
def kernel(x, single_w, pair_w):
    a = pair_w.reshape(L, L, V, V).transpose(0, 2, 1, 3).reshape(D, D)
    oh = jax.nn.one_hot(x, V, dtype=jnp.float32)
    row19 = jnp.broadcast_to(
        (jnp.arange(V) == 19).astype(jnp.float32)[None, None, :], (1, L, V))
    xoh = jnp.concatenate([oh, row19], axis=0).reshape(B + 1, D)
    xoh = jnp.pad(xoh, ((0, BP - (B + 1)), (0, 0)))
    swf = single_w.reshape(1, D)

    out = pl.pallas_call(
        _mrf_kernel,
        grid=(NBLK,),
        in_specs=[
            pl.BlockSpec((BLK, D), lambda ib: (ib, 0)),
            pl.BlockSpec((BP, D), lambda ib: (0, 0)),
            pl.BlockSpec((BP, BLK), lambda ib: (0, ib)),
            pl.BlockSpec((1, BLK), lambda ib: (0, ib)),
            pl.BlockSpec((1, D), lambda ib: (0, 0)),
        ],
        out_specs=pl.BlockSpec((1, 1), lambda ib: (0, 0),
                               memory_space=pltpu.MemorySpace.SMEM),
        out_shape=jax.ShapeDtypeStruct((1, 1), jnp.float32),
        scratch_shapes=[
            pltpu.VMEM((BP, L), jnp.float32),
            pltpu.VMEM((BP, L), jnp.float32),
            pltpu.SMEM((1, 1), jnp.float32),
            pltpu.SMEM((1, 1), jnp.float32),
        ],
        compiler_params=pltpu.CompilerParams(
            dimension_semantics=("arbitrary",),
        ),
    )(a, xoh, xoh, swf, swf)
    return out[0, 0]
